# baseline (speedup 1.0000x reference)
import math

import jax
import jax.numpy as jnp
from jax import lax
from jax.experimental import pallas as pl
from jax.experimental.pallas import tpu as pltpu
from jax.experimental.pallas import tpu_sc as plsc

LEVELS = 8
BASE_RES = 16.0
MAX_RES = 2048.0
FEAT = 2
MAX_PARAMS = 2 ** 19
LOG_B = math.log(MAX_RES / BASE_RES) / (LEVELS - 1)

_OFFS = []
_SIZES = []
_RES = []
_SCALES = []
_off = 0
_FIRST_HASHED = LEVELS
for _i in range(LEVELS):
    _scale = BASE_RES * math.exp(_i * LOG_B) - 1.0
    _res = int(math.ceil(_scale)) + 1
    _full = _res ** 3
    _full_aligned = ((_full + 7) // 8) * 8
    _sz = min(MAX_PARAMS, _full_aligned)
    _OFFS.append(_off)
    _SIZES.append(_sz)
    _RES.append(_res)
    _SCALES.append(_scale)
    if _full > _sz and _FIRST_HASHED == LEVELS:
        _FIRST_HASHED = _i
    _off += _sz
TOTAL_ROWS = _off
for _s in _SIZES:
    assert _s & (_s - 1) == 0, "level sizes must be powers of two"

P1 = 2654435761 - 2 ** 32
P2 = 805459861

B = 524288
NC, NS = 2, 16
NW = NC * NS
PB = B // NW
C = 128
NV = C // 16
NCHUNK = PB // C
N_LOCAL_LVL = 2
LOCAL_FLOATS = _OFFS[N_LOCAL_LVL] * FEAT
LOCAL_LC = N_LOCAL_LVL * 8
HBM_LC = (LEVELS - N_LOCAL_LVL) * 8
HBM_LVLS = tuple(range(N_LOCAL_LVL, LEVELS))


def _sc_body(pos_hbm, tabf_hbm, out_hbm,
             lvl01, pos_b, idxl_b, idx2_b, w_b, rows_b, out_b, sem_g):
    wid = lax.axis_index("s") * NC + lax.axis_index("c")
    tile_base = wid * PB

    pltpu.sync_copy(tabf_hbm.at[pl.ds(0, LOCAL_FLOATS)], lvl01)

    iota = lax.iota(jnp.int32, 16)

    def chunk(ci, _):
        base = tile_base + ci * C
        pltpu.sync_copy(pos_hbm.at[pl.ds(base, C)], pos_b)

        handles = []
        for l in list(HBM_LVLS) + list(range(N_LOCAL_LVL)):
            scale = _SCALES[l]
            mask = _SIZES[l] - 1
            off2 = _OFFS[l] * 2
            res = _RES[l]
            local = l < N_LOCAL_LVL

            def lvl_body(v, _, scale=scale, mask=mask, off2=off2, res=res,
                         l=l, local=local):
                rowi = v * 16 + iota
                px = plsc.load_gather(pos_b, [rowi, jnp.full((16,), 0, jnp.int32)]) * scale + 0.5
                py = plsc.load_gather(pos_b, [rowi, jnp.full((16,), 1, jnp.int32)]) * scale + 0.5
                pz = plsc.load_gather(pos_b, [rowi, jnp.full((16,), 2, jnp.int32)]) * scale + 0.5
                gx = px.astype(jnp.int32)
                gy = py.astype(jnp.int32)
                gz = pz.astype(jnp.int32)
                fx = px - gx.astype(jnp.float32)
                fy = py - gy.astype(jnp.float32)
                fz = pz - gz.astype(jnp.float32)
                if l < _FIRST_HASHED:
                    ax = (gx, gx + 1)
                    ay = (gy * res, gy * res + res)
                    az = (gz * (res * res), gz * (res * res) + res * res)
                    comb = lambda a, b: a + b
                else:
                    ax = (gx, gx + 1)
                    ay = (gy * P1, gy * P1 + P1)
                    az = (gz * P2, gz * P2 + P2)
                    comb = lax.bitwise_xor
                wx = (1.0 - fx, fx)
                wy = (1.0 - fy, fy)
                wz = (1.0 - fz, fz)
                wxy = [wx[0] * wy[0], wx[1] * wy[0], wx[0] * wy[1], wx[1] * wy[1]]
                for c in range(8):
                    cx, cy, cz = c & 1, (c >> 1) & 1, c >> 2
                    h = comb(comb(ax[cx], ay[cy]), az[cz])
                    i0 = ((h & mask) << 1) + off2
                    sl = pl.ds(v * 16, 16)
                    if local:
                        idxl_b[l * 8 + c, sl] = i0
                    else:
                        lc16 = (l - N_LOCAL_LVL) * 8 + c
                        idx2_b[2 * lc16, sl] = i0
                        idx2_b[2 * lc16 + 1, sl] = i0 + 1
                    w_b[l * 8 + c, sl] = wxy[cy * 2 + cx] * wz[cz]
                return _

            lax.fori_loop(0, NV, lvl_body, None)

            if not local:
                for c in range(8):
                    lc16 = (l - N_LOCAL_LVL) * 8 + c
                    for f in range(2):
                        handles.append(pltpu.async_copy(
                            tabf_hbm.at[idx2_b.at[2 * lc16 + f]],
                            rows_b.at[2 * lc16 + f], sem_g))

        for l in range(N_LOCAL_LVL):
            col0 = jnp.full((16,), 2 * l, jnp.int32)
            col1 = jnp.full((16,), 2 * l + 1, jnp.int32)

            def loc_body(v, _, l=l, col0=col0, col1=col1):
                sl = pl.ds(v * 16, 16)
                acc0 = jnp.zeros((16,), jnp.float32)
                acc1 = jnp.zeros((16,), jnp.float32)
                for c in range(8):
                    lc = l * 8 + c
                    i0 = idxl_b[lc, sl]
                    wv = w_b[lc, sl]
                    acc0 = acc0 + wv * plsc.load_gather(lvl01, [i0])
                    acc1 = acc1 + wv * plsc.load_gather(lvl01, [i0 + 1])
                rowv = v * 16 + iota
                plsc.store_scatter(out_b, [rowv, col0], acc0)
                plsc.store_scatter(out_b, [rowv, col1], acc1)
                return _

            lax.fori_loop(0, NV, loc_body, None)

        for h in handles:
            h.wait()

        for l in HBM_LVLS:
            col0 = jnp.full((16,), 2 * l, jnp.int32)
            col1 = jnp.full((16,), 2 * l + 1, jnp.int32)

            def hbm_body(v, _, l=l, col0=col0, col1=col1):
                sl = pl.ds(v * 16, 16)
                acc0 = jnp.zeros((16,), jnp.float32)
                acc1 = jnp.zeros((16,), jnp.float32)
                for c in range(8):
                    lc = l * 8 + c
                    lc16 = (l - N_LOCAL_LVL) * 8 + c
                    wv = w_b[lc, sl]
                    acc0 = acc0 + wv * rows_b[2 * lc16, sl]
                    acc1 = acc1 + wv * rows_b[2 * lc16 + 1, sl]
                rowv = v * 16 + iota
                plsc.store_scatter(out_b, [rowv, col0], acc0)
                plsc.store_scatter(out_b, [rowv, col1], acc1)
                return _

            lax.fori_loop(0, NV, hbm_body, None)

        pltpu.sync_copy(out_b, out_hbm.at[pl.ds(base, C)])
        return _

    lax.fori_loop(0, NCHUNK, chunk, None)


@jax.jit
def _encode_sc(positions, table_flat):
    mesh = plsc.VectorSubcoreMesh(core_axis_name="c", subcore_axis_name="s")
    return pl.kernel(
        _sc_body,
        out_type=jax.ShapeDtypeStruct((B, LEVELS * FEAT), jnp.float32),
        mesh=mesh,
        compiler_params=pltpu.CompilerParams(
            needs_layout_passes=False, use_tc_tiling_on_sc=False),
        scratch_types=[
            pltpu.VMEM((LOCAL_FLOATS,), jnp.float32),
            pltpu.VMEM((C, 3), jnp.float32),
            pltpu.VMEM((LOCAL_LC, C), jnp.int32),
            pltpu.VMEM((2 * HBM_LC, C), jnp.int32),
            pltpu.VMEM((LEVELS * 8, C), jnp.float32),
            pltpu.VMEM((2 * HBM_LC, C), jnp.float32),
            pltpu.VMEM((C, LEVELS * FEAT), jnp.float32),
            pltpu.SemaphoreType.DMA,
        ],
    )(positions, table_flat)


def kernel(positions, hash_table):
    return _encode_sc(positions, hash_table)


# speedup vs baseline: 1.4929x; 1.4929x over previous
import math

import jax
import jax.numpy as jnp
from jax import lax
from jax.experimental import pallas as pl
from jax.experimental.pallas import tpu as pltpu
from jax.experimental.pallas import tpu_sc as plsc

LEVELS = 8
BASE_RES = 16.0
MAX_RES = 2048.0
FEAT = 2
MAX_PARAMS = 2 ** 19
LOG_B = math.log(MAX_RES / BASE_RES) / (LEVELS - 1)

_OFFS = []
_SIZES = []
_RES = []
_SCALES = []
_off = 0
_FIRST_HASHED = LEVELS
for _i in range(LEVELS):
    _scale = BASE_RES * math.exp(_i * LOG_B) - 1.0
    _res = int(math.ceil(_scale)) + 1
    _full = _res ** 3
    _full_aligned = ((_full + 7) // 8) * 8
    _sz = min(MAX_PARAMS, _full_aligned)
    _OFFS.append(_off)
    _SIZES.append(_sz)
    _RES.append(_res)
    _SCALES.append(_scale)
    if _full > _sz and _FIRST_HASHED == LEVELS:
        _FIRST_HASHED = _i
    _off += _sz
TOTAL_ROWS = _off
for _s in _SIZES:
    assert _s & (_s - 1) == 0, "level sizes must be powers of two"

P1 = 2654435761 - 2 ** 32
P2 = 805459861

B = 524288
NC, NS = 2, 16
NW = NC * NS
PB = B // NW
C = 128
NV = C // 16
NCHUNK = PB // C
N_LOCAL_LVL = 2
LOCAL_FLOATS = _OFFS[N_LOCAL_LVL] * FEAT
LOCAL_LC = N_LOCAL_LVL * 8
HBM_LC = (LEVELS - N_LOCAL_LVL) * 8
HBM_LVLS = tuple(range(N_LOCAL_LVL, LEVELS))


def _sc_body(pos_hbm, tabf_hbm, out_hbm,
             lvl01, pos_b, idxl_b, idx2_b, w_b, rows_b, out_b, sem_g):
    wid = lax.axis_index("s") * NC + lax.axis_index("c")
    tile_base = wid * PB

    pltpu.sync_copy(tabf_hbm.at[pl.ds(0, LOCAL_FLOATS)], lvl01)

    iota = lax.iota(jnp.int32, 16)

    def chunk(ci, _):
        base = tile_base + ci * C
        pltpu.sync_copy(pos_hbm.at[:, pl.ds(base, C)], pos_b)

        handles = []
        for l in list(HBM_LVLS) + list(range(N_LOCAL_LVL)):
            scale = _SCALES[l]
            mask = _SIZES[l] - 1
            off2 = _OFFS[l] * 2
            res = _RES[l]
            local = l < N_LOCAL_LVL

            def lvl_body(v, _, scale=scale, mask=mask, off2=off2, res=res,
                         l=l, local=local):
                psl = pl.ds(v * 16, 16)
                px = pos_b[0, psl] * scale + 0.5
                py = pos_b[1, psl] * scale + 0.5
                pz = pos_b[2, psl] * scale + 0.5
                gx = px.astype(jnp.int32)
                gy = py.astype(jnp.int32)
                gz = pz.astype(jnp.int32)
                fx = px - gx.astype(jnp.float32)
                fy = py - gy.astype(jnp.float32)
                fz = pz - gz.astype(jnp.float32)
                if l < _FIRST_HASHED:
                    ax = (gx, gx + 1)
                    ay = (gy * res, gy * res + res)
                    az = (gz * (res * res), gz * (res * res) + res * res)
                    comb = lambda a, b: a + b
                else:
                    ax = (gx, gx + 1)
                    ay = (gy * P1, gy * P1 + P1)
                    az = (gz * P2, gz * P2 + P2)
                    comb = lax.bitwise_xor
                wx = (1.0 - fx, fx)
                wy = (1.0 - fy, fy)
                wz = (1.0 - fz, fz)
                wxy = [wx[0] * wy[0], wx[1] * wy[0], wx[0] * wy[1], wx[1] * wy[1]]
                for c in range(8):
                    cx, cy, cz = c & 1, (c >> 1) & 1, c >> 2
                    h = comb(comb(ax[cx], ay[cy]), az[cz])
                    i0 = ((h & mask) << 1) + off2
                    sl = pl.ds(v * 16, 16)
                    if local:
                        idxl_b[l * 8 + c, sl] = i0
                    else:
                        lc16 = (l - N_LOCAL_LVL) * 8 + c
                        idx2_b[2 * lc16, sl] = i0
                        idx2_b[2 * lc16 + 1, sl] = i0 + 1
                    w_b[l * 8 + c, sl] = wxy[cy * 2 + cx] * wz[cz]
                return _

            lax.fori_loop(0, NV, lvl_body, None)

            if not local:
                for c in range(8):
                    lc16 = (l - N_LOCAL_LVL) * 8 + c
                    for f in range(2):
                        handles.append(pltpu.async_copy(
                            tabf_hbm.at[idx2_b.at[2 * lc16 + f]],
                            rows_b.at[2 * lc16 + f], sem_g))

        for l in range(N_LOCAL_LVL):

            def loc_body(v, _, l=l):
                sl = pl.ds(v * 16, 16)
                acc0 = jnp.zeros((16,), jnp.float32)
                acc1 = jnp.zeros((16,), jnp.float32)
                for c in range(8):
                    lc = l * 8 + c
                    i0 = idxl_b[lc, sl]
                    wv = w_b[lc, sl]
                    acc0 = acc0 + wv * plsc.load_gather(lvl01, [i0])
                    acc1 = acc1 + wv * plsc.load_gather(lvl01, [i0 + 1])
                out_b[2 * l, sl] = acc0
                out_b[2 * l + 1, sl] = acc1
                return _

            lax.fori_loop(0, NV, loc_body, None)

        for h in handles:
            h.wait()

        for l in HBM_LVLS:

            def hbm_body(v, _, l=l):
                sl = pl.ds(v * 16, 16)
                acc0 = jnp.zeros((16,), jnp.float32)
                acc1 = jnp.zeros((16,), jnp.float32)
                for c in range(8):
                    lc = l * 8 + c
                    lc16 = (l - N_LOCAL_LVL) * 8 + c
                    wv = w_b[lc, sl]
                    acc0 = acc0 + wv * rows_b[2 * lc16, sl]
                    acc1 = acc1 + wv * rows_b[2 * lc16 + 1, sl]
                out_b[2 * l, sl] = acc0
                out_b[2 * l + 1, sl] = acc1
                return _

            lax.fori_loop(0, NV, hbm_body, None)

        pltpu.sync_copy(out_b, out_hbm.at[:, pl.ds(base, C)])
        return _

    lax.fori_loop(0, NCHUNK, chunk, None)


@jax.jit
def _encode_sc(positions, table_flat):
    mesh = plsc.VectorSubcoreMesh(core_axis_name="c", subcore_axis_name="s")
    return pl.kernel(
        _sc_body,
        out_type=jax.ShapeDtypeStruct((LEVELS * FEAT, B), jnp.float32),
        mesh=mesh,
        compiler_params=pltpu.CompilerParams(
            needs_layout_passes=False, use_tc_tiling_on_sc=False),
        scratch_types=[
            pltpu.VMEM((LOCAL_FLOATS,), jnp.float32),
            pltpu.VMEM((3, C), jnp.float32),
            pltpu.VMEM((LOCAL_LC, C), jnp.int32),
            pltpu.VMEM((2 * HBM_LC, C), jnp.int32),
            pltpu.VMEM((LEVELS * 8, C), jnp.float32),
            pltpu.VMEM((2 * HBM_LC, C), jnp.float32),
            pltpu.VMEM((LEVELS * FEAT, C), jnp.float32),
            pltpu.SemaphoreType.DMA,
        ],
    )(positions, table_flat)


def kernel(positions, hash_table):
    return _encode_sc(positions.T, hash_table).T


# speedup vs baseline: 2.1027x; 1.4085x over previous
import math

import jax
import jax.numpy as jnp
from jax import lax
from jax.experimental import pallas as pl
from jax.experimental.pallas import tpu as pltpu
from jax.experimental.pallas import tpu_sc as plsc

LEVELS = 8
BASE_RES = 16.0
MAX_RES = 2048.0
FEAT = 2
MAX_PARAMS = 2 ** 19
LOG_B = math.log(MAX_RES / BASE_RES) / (LEVELS - 1)

_OFFS = []
_SIZES = []
_RES = []
_SCALES = []
_off = 0
_FIRST_HASHED = LEVELS
for _i in range(LEVELS):
    _scale = BASE_RES * math.exp(_i * LOG_B) - 1.0
    _res = int(math.ceil(_scale)) + 1
    _full = _res ** 3
    _full_aligned = ((_full + 7) // 8) * 8
    _sz = min(MAX_PARAMS, _full_aligned)
    _OFFS.append(_off)
    _SIZES.append(_sz)
    _RES.append(_res)
    _SCALES.append(_scale)
    if _full > _sz and _FIRST_HASHED == LEVELS:
        _FIRST_HASHED = _i
    _off += _sz
TOTAL_ROWS = _off
for _s in _SIZES:
    assert _s & (_s - 1) == 0, "level sizes must be powers of two"

P1 = 2654435761 - 2 ** 32
P2 = 805459861

B = 524288
NC, NS = 2, 16
NW = NC * NS
PB = B // NW
C = 64
NV = C // 16
NCHUNK = PB // C
N_LOCAL_LVL = 2
LOCAL_FLOATS = _OFFS[N_LOCAL_LVL] * FEAT
LOCAL_LC = N_LOCAL_LVL * 8
HBM_LC = (LEVELS - N_LOCAL_LVL) * 8
HBM_LVLS = tuple(range(N_LOCAL_LVL, LEVELS))


def _sc_body(pos_hbm, tab8_hbm, out_hbm,
             lvl01, pos_b, idxl_b, idx2_b, sub_b, w_b, rows_b, out_b, sem_g):
    wid = lax.axis_index("s") * NC + lax.axis_index("c")
    tile_base = wid * PB

    pltpu.sync_copy(tab8_hbm.at[pl.ds(0, LOCAL_FLOATS // 8)], lvl01)

    iota = lax.iota(jnp.int32, 16)

    def chunk(ci, _):
        base = tile_base + ci * C
        pltpu.sync_copy(pos_hbm.at[:, pl.ds(base, C)], pos_b)

        handles = []
        for l in list(HBM_LVLS) + list(range(N_LOCAL_LVL)):
            scale = _SCALES[l]
            mask = _SIZES[l] - 1
            off = _OFFS[l]
            res = _RES[l]
            local = l < N_LOCAL_LVL

            def lvl_body(v, _, scale=scale, mask=mask, off=off, res=res,
                         l=l, local=local):
                psl = pl.ds(v * 16, 16)
                px = pos_b[0, psl] * scale + 0.5
                py = pos_b[1, psl] * scale + 0.5
                pz = pos_b[2, psl] * scale + 0.5
                gx = px.astype(jnp.int32)
                gy = py.astype(jnp.int32)
                gz = pz.astype(jnp.int32)
                fx = px - gx.astype(jnp.float32)
                fy = py - gy.astype(jnp.float32)
                fz = pz - gz.astype(jnp.float32)
                if l < _FIRST_HASHED:
                    ax = (gx, gx + 1)
                    ay = (gy * res, gy * res + res)
                    az = (gz * (res * res), gz * (res * res) + res * res)
                    comb = lambda a, b: a + b
                else:
                    ax = (gx, gx + 1)
                    ay = (gy * P1, gy * P1 + P1)
                    az = (gz * P2, gz * P2 + P2)
                    comb = lax.bitwise_xor
                wx = (1.0 - fx, fx)
                wy = (1.0 - fy, fy)
                wz = (1.0 - fz, fz)
                wxy = [wx[0] * wy[0], wx[1] * wy[0], wx[0] * wy[1], wx[1] * wy[1]]
                for c in range(8):
                    cx, cy, cz = c & 1, (c >> 1) & 1, c >> 2
                    h = comb(comb(ax[cx], ay[cy]), az[cz])
                    e = (h & mask) + off
                    sl = pl.ds(v * 16, 16)
                    if local:
                        idxl_b[l * 8 + c, sl] = e
                    else:
                        lc16 = (l - N_LOCAL_LVL) * 8 + c
                        idx2_b[lc16, sl] = e >> 2
                        sub_b[lc16, sl] = (e & 3) << 1
                    w_b[l * 8 + c, sl] = wxy[cy * 2 + cx] * wz[cz]
                return _

            lax.fori_loop(0, NV, lvl_body, None)

            if not local:
                for c in range(8):
                    lc16 = (l - N_LOCAL_LVL) * 8 + c
                    handles.append(pltpu.async_copy(
                        tab8_hbm.at[idx2_b.at[lc16]],
                        rows_b.at[lc16], sem_g))

        for l in range(N_LOCAL_LVL):

            def loc_body(v, _, l=l):
                sl = pl.ds(v * 16, 16)
                acc0 = jnp.zeros((16,), jnp.float32)
                acc1 = jnp.zeros((16,), jnp.float32)
                for c in range(8):
                    lc = l * 8 + c
                    ev = idxl_b[lc, sl]
                    rv = ev >> 2
                    sv = (ev & 3) << 1
                    wv = w_b[lc, sl]
                    acc0 = acc0 + wv * plsc.load_gather(lvl01, [rv, sv])
                    acc1 = acc1 + wv * plsc.load_gather(lvl01, [rv, sv + 1])
                out_b[2 * l, sl] = acc0
                out_b[2 * l + 1, sl] = acc1
                return _

            lax.fori_loop(0, NV, loc_body, None)

        for h in handles:
            h.wait()

        for l in HBM_LVLS:

            def hbm_body(v, _, l=l):
                sl = pl.ds(v * 16, 16)
                posv = v * 16 + iota
                acc0 = jnp.zeros((16,), jnp.float32)
                acc1 = jnp.zeros((16,), jnp.float32)
                for c in range(8):
                    lc = l * 8 + c
                    lc16 = (l - N_LOCAL_LVL) * 8 + c
                    slot = jnp.full((16,), lc16, jnp.int32)
                    wv = w_b[lc, sl]
                    subv = sub_b[lc16, sl]
                    acc0 = acc0 + wv * plsc.load_gather(rows_b, [slot, posv, subv])
                    acc1 = acc1 + wv * plsc.load_gather(rows_b, [slot, posv, subv + 1])
                out_b[2 * l, sl] = acc0
                out_b[2 * l + 1, sl] = acc1
                return _

            lax.fori_loop(0, NV, hbm_body, None)

        pltpu.sync_copy(out_b, out_hbm.at[:, pl.ds(base, C)])
        return _

    lax.fori_loop(0, NCHUNK, chunk, None)


@jax.jit
def _encode_sc(positions, table8):
    mesh = plsc.VectorSubcoreMesh(core_axis_name="c", subcore_axis_name="s")
    return pl.kernel(
        _sc_body,
        out_type=jax.ShapeDtypeStruct((LEVELS * FEAT, B), jnp.float32),
        mesh=mesh,
        compiler_params=pltpu.CompilerParams(
            needs_layout_passes=False, use_tc_tiling_on_sc=False),
        scratch_types=[
            pltpu.VMEM((LOCAL_FLOATS // 8, 8), jnp.float32),
            pltpu.VMEM((3, C), jnp.float32),
            pltpu.VMEM((LOCAL_LC, C), jnp.int32),
            pltpu.VMEM((HBM_LC, C), jnp.int32),
            pltpu.VMEM((HBM_LC, C), jnp.int32),
            pltpu.VMEM((LEVELS * 8, C), jnp.float32),
            pltpu.VMEM((HBM_LC, C, 8), jnp.float32),
            pltpu.VMEM((LEVELS * FEAT, C), jnp.float32),
            pltpu.SemaphoreType.DMA,
        ],
    )(positions, table8)


def kernel(positions, hash_table):
    return _encode_sc(positions.T, hash_table.reshape(-1, 8)).T


# speedup vs baseline: 2.1073x; 1.0021x over previous
import math

import jax
import jax.numpy as jnp
from jax import lax
from jax.experimental import pallas as pl
from jax.experimental.pallas import tpu as pltpu
from jax.experimental.pallas import tpu_sc as plsc

LEVELS = 8
BASE_RES = 16.0
MAX_RES = 2048.0
FEAT = 2
MAX_PARAMS = 2 ** 19
LOG_B = math.log(MAX_RES / BASE_RES) / (LEVELS - 1)

_OFFS = []
_SIZES = []
_RES = []
_SCALES = []
_off = 0
_FIRST_HASHED = LEVELS
for _i in range(LEVELS):
    _scale = BASE_RES * math.exp(_i * LOG_B) - 1.0
    _res = int(math.ceil(_scale)) + 1
    _full = _res ** 3
    _full_aligned = ((_full + 7) // 8) * 8
    _sz = min(MAX_PARAMS, _full_aligned)
    _OFFS.append(_off)
    _SIZES.append(_sz)
    _RES.append(_res)
    _SCALES.append(_scale)
    if _full > _sz and _FIRST_HASHED == LEVELS:
        _FIRST_HASHED = _i
    _off += _sz
TOTAL_ROWS = _off
for _s in _SIZES:
    assert _s & (_s - 1) == 0, "level sizes must be powers of two"

P1 = 2654435761 - 2 ** 32
P2 = 805459861

B = 524288
NC, NS = 2, 16
NW = NC * NS
PB = B // NW
C = 64
NV = C // 16
NCHUNK = PB // C
N_LOCAL_LVL = 2
LOCAL_FLOATS = _OFFS[N_LOCAL_LVL] * FEAT
LOCAL_LC = N_LOCAL_LVL * 8
HBM_LC = (LEVELS - N_LOCAL_LVL) * 8
HBM_LVLS = tuple(range(N_LOCAL_LVL, LEVELS))


def _sc_body(pos_hbm, tab8_hbm, out_hbm,
             lvl01, pos_b, idxl_b, idx2_b, sub_b, w_b, rows_b, out_b, sem_g):
    wid = lax.axis_index("s") * NC + lax.axis_index("c")
    tile_base = wid * PB

    pltpu.sync_copy(tab8_hbm.at[pl.ds(0, LOCAL_FLOATS // 8)], lvl01)

    iota = lax.iota(jnp.int32, 16)

    def chunk(ci, _):
        base = tile_base + ci * C
        pltpu.sync_copy(pos_hbm.at[:, pl.ds(base, C)], pos_b)

        handles = {l: [] for l in HBM_LVLS}
        for l in list(HBM_LVLS) + list(range(N_LOCAL_LVL)):
            scale = _SCALES[l]
            mask = _SIZES[l] - 1
            off = _OFFS[l]
            res = _RES[l]
            local = l < N_LOCAL_LVL

            def lvl_body(v, _, scale=scale, mask=mask, off=off, res=res,
                         l=l, local=local):
                psl = pl.ds(v * 16, 16)
                px = pos_b[0, psl] * scale + 0.5
                py = pos_b[1, psl] * scale + 0.5
                pz = pos_b[2, psl] * scale + 0.5
                gx = px.astype(jnp.int32)
                gy = py.astype(jnp.int32)
                gz = pz.astype(jnp.int32)
                fx = px - gx.astype(jnp.float32)
                fy = py - gy.astype(jnp.float32)
                fz = pz - gz.astype(jnp.float32)
                if l < _FIRST_HASHED:
                    ax = (gx, gx + 1)
                    ay = (gy * res, gy * res + res)
                    az = (gz * (res * res), gz * (res * res) + res * res)
                    comb = lambda a, b: a + b
                else:
                    ax = (gx, gx + 1)
                    ay = (gy * P1, gy * P1 + P1)
                    az = (gz * P2, gz * P2 + P2)
                    comb = lax.bitwise_xor
                wx = (1.0 - fx, fx)
                wy = (1.0 - fy, fy)
                wz = (1.0 - fz, fz)
                wxy = [wx[0] * wy[0], wx[1] * wy[0], wx[0] * wy[1], wx[1] * wy[1]]
                for c in range(8):
                    cx, cy, cz = c & 1, (c >> 1) & 1, c >> 2
                    h = comb(comb(ax[cx], ay[cy]), az[cz])
                    e = (h & mask) + off
                    sl = pl.ds(v * 16, 16)
                    if local:
                        idxl_b[l * 8 + c, sl] = e
                    else:
                        lc16 = (l - N_LOCAL_LVL) * 8 + c
                        idx2_b[lc16, sl] = e >> 2
                        sub_b[lc16, sl] = (e & 3) << 1
                    w_b[l * 8 + c, sl] = wxy[cy * 2 + cx] * wz[cz]
                return _

            lax.fori_loop(0, NV, lvl_body, None)

            if not local:
                for c in range(8):
                    lc16 = (l - N_LOCAL_LVL) * 8 + c
                    handles[l].append(pltpu.async_copy(
                        tab8_hbm.at[idx2_b.at[lc16]],
                        rows_b.at[lc16], sem_g))

        for l in range(N_LOCAL_LVL):

            def loc_body(v, _, l=l):
                sl = pl.ds(v * 16, 16)
                acc0 = jnp.zeros((16,), jnp.float32)
                acc1 = jnp.zeros((16,), jnp.float32)
                for c in range(8):
                    lc = l * 8 + c
                    ev = idxl_b[lc, sl]
                    rv = ev >> 2
                    sv = (ev & 3) << 1
                    wv = w_b[lc, sl]
                    acc0 = acc0 + wv * plsc.load_gather(lvl01, [rv, sv])
                    acc1 = acc1 + wv * plsc.load_gather(lvl01, [rv, sv + 1])
                out_b[2 * l, sl] = acc0
                out_b[2 * l + 1, sl] = acc1
                return _

            lax.fori_loop(0, NV, loc_body, None)

        for l in HBM_LVLS:
            for h in handles[l]:
                h.wait()

            def hbm_body(v, _, l=l):
                sl = pl.ds(v * 16, 16)
                posv = v * 16 + iota
                acc0 = jnp.zeros((16,), jnp.float32)
                acc1 = jnp.zeros((16,), jnp.float32)
                for c in range(8):
                    lc = l * 8 + c
                    lc16 = (l - N_LOCAL_LVL) * 8 + c
                    slot = jnp.full((16,), lc16, jnp.int32)
                    wv = w_b[lc, sl]
                    subv = sub_b[lc16, sl]
                    acc0 = acc0 + wv * plsc.load_gather(rows_b, [slot, posv, subv])
                    acc1 = acc1 + wv * plsc.load_gather(rows_b, [slot, posv, subv + 1])
                out_b[2 * l, sl] = acc0
                out_b[2 * l + 1, sl] = acc1
                return _

            lax.fori_loop(0, NV, hbm_body, None)

        pltpu.sync_copy(out_b, out_hbm.at[:, pl.ds(base, C)])
        return _

    lax.fori_loop(0, NCHUNK, chunk, None)


@jax.jit
def _encode_sc(positions, table8):
    mesh = plsc.VectorSubcoreMesh(core_axis_name="c", subcore_axis_name="s")
    return pl.kernel(
        _sc_body,
        out_type=jax.ShapeDtypeStruct((LEVELS * FEAT, B), jnp.float32),
        mesh=mesh,
        compiler_params=pltpu.CompilerParams(
            needs_layout_passes=False, use_tc_tiling_on_sc=False),
        scratch_types=[
            pltpu.VMEM((LOCAL_FLOATS // 8, 8), jnp.float32),
            pltpu.VMEM((3, C), jnp.float32),
            pltpu.VMEM((LOCAL_LC, C), jnp.int32),
            pltpu.VMEM((HBM_LC, C), jnp.int32),
            pltpu.VMEM((HBM_LC, C), jnp.int32),
            pltpu.VMEM((LEVELS * 8, C), jnp.float32),
            pltpu.VMEM((HBM_LC, C, 8), jnp.float32),
            pltpu.VMEM((LEVELS * FEAT, C), jnp.float32),
            pltpu.SemaphoreType.DMA,
        ],
    )(positions, table8)


def kernel(positions, hash_table):
    return _encode_sc(positions.T, hash_table.reshape(-1, 8)).T
